# baseline (speedup 1.0000x reference)
import functools

import jax
import jax.numpy as jnp
from jax import lax
from jax.experimental import pallas as pl
from jax.experimental.pallas import tpu as pltpu
from jax.experimental.pallas import tpu_sc as plsc

_N = 10000
_E = 320000
_D = 128
_M = 4
_TEMP = 10.0

_NC = 2
_NS = 16
_NW = _NC * _NS
_CHUNK = 128
_CPW = 79
_TOTCH = _NW * _CPW
_EPAD = _TOTCH * _CHUNK
_NPAD = 10240
_RPT = _NPAD // _NS
_DB = _NPAD // _CHUNK


def _sc_agg_body(src_hbm, dst_hbm, x_hbm, zeros_hbm, zflat_hbm,
                 agg_out, deg_out,
                 src_v, dst_v, rows_v, deg_v, agg_s, gsem):
    cid = lax.axis_index("c")
    sid = lax.axis_index("s")
    wid = sid * _NC + cid

    pltpu.sync_copy(zeros_hbm, agg_s.at[pl.ds(sid * _RPT, _RPT)])
    pltpu.sync_copy(zflat_hbm, deg_v)
    pltpu.sync_copy(src_hbm.at[wid], src_v)
    pltpu.sync_copy(dst_hbm.at[wid], dst_v)
    plsc.subcore_barrier()

    _H = 2 * _CPW

    def _g_start(h, b):
        j, k = h >> 1, h & 1
        pltpu.async_copy(
            x_hbm.at[src_v.at[j].at[pl.ds(k * 64, 64)]],
            rows_v.at[b], gsem)

    def _g_wait(h, b):
        j, k = h >> 1, h & 1
        pltpu.make_async_copy(
            x_hbm.at[src_v.at[j].at[pl.ds(k * 64, 64)]],
            rows_v.at[b], gsem).wait()

    _g_start(0, 0)

    ones16 = jnp.full((16,), 1.0, jnp.float32)

    def _hist(t, carry):
        j = t // (_CHUNK // 16)
        k = t % (_CHUNK // 16)
        v = dst_v[j, pl.ds(k * 16, 16)]
        plsc.addupdate_scatter(deg_v, [v], ones16)
        return carry

    lax.fori_loop(0, _CPW * (_CHUNK // 16), _hist, 0)

    def _edge_step(h, carry):
        b = h & 1
        _g_wait(h, b)

        @pl.when(h + 1 < _H)
        def _prefetch():
            _g_start(h + 1, 1 - b)

        j, k = h >> 1, h & 1
        pltpu.sync_copy(rows_v.at[b],
                        agg_s.at[dst_v.at[j].at[pl.ds(k * 64, 64)]],
                        add=True)
        return carry

    lax.fori_loop(0, _H, _edge_step, 0)

    pltpu.sync_copy(deg_v, deg_out.at[cid].at[sid])
    plsc.subcore_barrier()

    pltpu.sync_copy(agg_s.at[pl.ds(sid * _RPT, _RPT)],
                    agg_out.at[cid].at[pl.ds(sid * _RPT, _RPT)])


@functools.cache
def _sc_agg():
  return functools.partial(
    pl.kernel,
    mesh=plsc.VectorSubcoreMesh(core_axis_name="c", subcore_axis_name="s",
                                num_cores=_NC, num_subcores=_NS),
    out_type=[
        jax.ShapeDtypeStruct((_NC, _NPAD, _D), jnp.float32),
        jax.ShapeDtypeStruct((_NC, _NS, _NPAD), jnp.float32),
    ],
    scratch_types=[
        pltpu.VMEM((_CPW, _CHUNK), jnp.int32),
        pltpu.VMEM((_CPW, _CHUNK), jnp.int32),
        pltpu.VMEM((2, _CHUNK // 2, _D), jnp.float32),
        pltpu.VMEM((_NPAD,), jnp.float32),
        pltpu.VMEM_SHARED((_NPAD, _D), jnp.float32),
        pltpu.SemaphoreType.DMA,
    ],
    compiler_params=pltpu.CompilerParams(needs_layout_passes=False),
  )(_sc_agg_body)


_RB = 512
_QB = _RB // _CHUNK


def _dense_body(x_ref, agg_ref, deg_ref, wqc_ref, bc_ref, v_ref, o_ref):
    x = x_ref[...]
    logits = jnp.dot(x, wqc_ref[...], preferred_element_type=jnp.float32)
    logits = logits + bc_ref[...]
    mx = jnp.max(logits, axis=-1, keepdims=True)
    ex = jnp.exp(logits - mx)
    choice = ex / jnp.sum(ex, axis=-1, keepdims=True)

    agg = agg_ref[0] + agg_ref[1]
    deg = jnp.sum(deg_ref[...], axis=(0, 2))
    recip = 1.0 / jnp.maximum(deg, 1.0)
    rows = lax.broadcasted_iota(jnp.int32, (_CHUNK, _CHUNK), 0)
    cols = lax.broadcasted_iota(jnp.int32, (_CHUNK, _CHUNK), 1)
    eye = rows == cols
    parts = []
    for q in range(_QB):
        diag = jnp.where(eye,
                         jnp.broadcast_to(recip[q:q + 1], (_CHUNK, _CHUNK)),
                         0.0)
        parts.append(jnp.dot(diag, agg[q * _CHUNK:(q + 1) * _CHUNK],
                             preferred_element_type=jnp.float32))
    aggm = jnp.concatenate(parts, axis=0)

    acc = x
    for m in range(_M):
        tm = jnp.dot(aggm, v_ref[m], preferred_element_type=jnp.float32)
        acc = acc + choice[:, m:m + 1] * tm
    o_ref[...] = jnp.maximum(acc, 0.0)


def _dense_call(x, agg2, deg4, wqc, bc, V):
    grid = (_N + _RB - 1) // _RB
    return pl.pallas_call(
        _dense_body,
        grid=(grid,),
        in_specs=[
            pl.BlockSpec((_RB, _D), lambda i: (i, 0)),
            pl.BlockSpec((_NC, _RB, _D), lambda i: (0, i, 0)),
            pl.BlockSpec((_NW, _QB, 1, _CHUNK), lambda i: (0, i, 0, 0)),
            pl.BlockSpec((_D, _M), lambda i: (0, 0)),
            pl.BlockSpec((1, _M), lambda i: (0, 0)),
            pl.BlockSpec((_M, _D, _D), lambda i: (0, 0, 0)),
        ],
        out_specs=pl.BlockSpec((_RB, _D), lambda i: (i, 0)),
        out_shape=jax.ShapeDtypeStruct((_N, _D), jnp.float32),
    )(x, agg2, deg4, wqc, bc, V)


def kernel(x, edge_index, Wq, bq, Wcode, V):
    src = edge_index[0]
    dst = edge_index[1]
    pad = _EPAD - _E
    src_p = jnp.concatenate(
        [src, jnp.zeros((pad,), jnp.int32)]).reshape(_NW, _CPW, _CHUNK)
    dump = _N + jnp.arange(pad, dtype=jnp.int32) % (_NPAD - _N)
    dst_p = jnp.concatenate([dst, dump]).reshape(_NW, _CPW, _CHUNK)
    zeros = jnp.zeros((_RPT, _D), jnp.float32)
    zflat = jnp.zeros((_NPAD,), jnp.float32)

    agg2, deg2 = _sc_agg()(src_p, dst_p, x, zeros, zflat)

    wqc = (Wq @ Wcode.T) / _TEMP
    bc = (bq[None, :] @ Wcode.T) / _TEMP

    deg4 = deg2.reshape(_NW, _DB, 1, _CHUNK)
    return _dense_call(x, agg2, deg4, wqc, bc, V)


# speedup vs baseline: 1.0051x; 1.0051x over previous
import functools

import jax
import jax.numpy as jnp
from jax import lax
from jax.experimental import pallas as pl
from jax.experimental.pallas import tpu as pltpu
from jax.experimental.pallas import tpu_sc as plsc

_N = 10000
_E = 320000
_D = 128
_M = 4
_TEMP = 10.0

_NC = 2
_NS = 16
_NW = _NC * _NS
_CHUNK = 128
_CPW = 79
_TOTCH = _NW * _CPW
_EPAD = _TOTCH * _CHUNK
_NPAD = 10240
_RPT = _NPAD // _NS
_DB = _NPAD // _CHUNK


def _sc_agg_body(src_hbm, dst_hbm, x_hbm, zeros_hbm, zflat_hbm,
                 agg_out, deg_out,
                 src_v, dst_v, rows_v, deg_v, agg_s, gsem):
    cid = lax.axis_index("c")
    sid = lax.axis_index("s")
    wid = sid * _NC + cid

    pltpu.sync_copy(zeros_hbm, agg_s.at[pl.ds(sid * _RPT, _RPT)])
    pltpu.sync_copy(zflat_hbm, deg_v)
    pltpu.sync_copy(src_hbm.at[wid], src_v)
    pltpu.sync_copy(dst_hbm.at[wid], dst_v)
    plsc.subcore_barrier()

    _H = 2 * _CPW

    def _g_start(h, b):
        j, k = h >> 1, h & 1
        pltpu.async_copy(
            x_hbm.at[src_v.at[j].at[pl.ds(k * 64, 64)]],
            rows_v.at[b], gsem)

    def _g_wait(h, b):
        j, k = h >> 1, h & 1
        pltpu.make_async_copy(
            x_hbm.at[src_v.at[j].at[pl.ds(k * 64, 64)]],
            rows_v.at[b], gsem).wait()

    _g_start(0, 0)

    ones16 = jnp.full((16,), 1.0, jnp.float32)

    def _hist(t, carry):
        j = t // (_CHUNK // 16)
        k = t % (_CHUNK // 16)
        v = dst_v[j, pl.ds(k * 16, 16)]
        plsc.addupdate_scatter(deg_v, [v], ones16)
        return carry

    lax.fori_loop(0, _CPW * (_CHUNK // 16), _hist, 0)

    def _edge_step(h, carry):
        b = h & 1
        _g_wait(h, b)

        @pl.when(h + 1 < _H)
        def _prefetch():
            _g_start(h + 1, 1 - b)

        j, k = h >> 1, h & 1
        pltpu.sync_copy(rows_v.at[b],
                        agg_s.at[dst_v.at[j].at[pl.ds(k * 64, 64)]],
                        add=True)
        return carry

    lax.fori_loop(0, _H, _edge_step, 0)

    pltpu.sync_copy(deg_v, deg_out.at[cid].at[sid])
    plsc.subcore_barrier()

    pltpu.sync_copy(agg_s.at[pl.ds(sid * _RPT, _RPT)],
                    agg_out.at[cid].at[pl.ds(sid * _RPT, _RPT)])


@functools.cache
def _sc_agg():
  return functools.partial(
    pl.kernel,
    mesh=plsc.VectorSubcoreMesh(core_axis_name="c", subcore_axis_name="s",
                                num_cores=_NC, num_subcores=_NS),
    out_type=[
        jax.ShapeDtypeStruct((_NC, _NPAD, _D), jnp.float32),
        jax.ShapeDtypeStruct((_NC, _NS, _NPAD), jnp.float32),
    ],
    scratch_types=[
        pltpu.VMEM((_CPW, _CHUNK), jnp.int32),
        pltpu.VMEM((_CPW, _CHUNK), jnp.int32),
        pltpu.VMEM((2, _CHUNK // 2, _D), jnp.float32),
        pltpu.VMEM((_NPAD,), jnp.float32),
        pltpu.VMEM_SHARED((_NPAD, _D), jnp.float32),
        pltpu.SemaphoreType.DMA,
    ],
    compiler_params=pltpu.CompilerParams(needs_layout_passes=False),
  )(_sc_agg_body)


_RB = 1024
_QB = _RB // _CHUNK


def _dense_body(x_ref, agg_ref, deg_ref, wqc_ref, bc_ref, v_ref, o_ref):
    x = x_ref[...]
    logits = jnp.dot(x, wqc_ref[...], preferred_element_type=jnp.float32)
    logits = logits + bc_ref[...]
    mx = jnp.max(logits, axis=-1, keepdims=True)
    ex = jnp.exp(logits - mx)
    choice = ex / jnp.sum(ex, axis=-1, keepdims=True)

    agg = agg_ref[0] + agg_ref[1]
    deg = jnp.sum(deg_ref[...], axis=(0, 2))
    recip = 1.0 / jnp.maximum(deg, 1.0)
    rows = lax.broadcasted_iota(jnp.int32, (_CHUNK, _CHUNK), 0)
    cols = lax.broadcasted_iota(jnp.int32, (_CHUNK, _CHUNK), 1)
    eye = rows == cols
    parts = []
    for q in range(_QB):
        diag = jnp.where(eye,
                         jnp.broadcast_to(recip[q:q + 1], (_CHUNK, _CHUNK)),
                         0.0)
        parts.append(jnp.dot(diag, agg[q * _CHUNK:(q + 1) * _CHUNK],
                             preferred_element_type=jnp.float32))
    aggm = jnp.concatenate(parts, axis=0)

    acc = x
    for m in range(_M):
        tm = jnp.dot(aggm, v_ref[m], preferred_element_type=jnp.float32)
        acc = acc + choice[:, m:m + 1] * tm
    o_ref[...] = jnp.maximum(acc, 0.0)


def _dense_call(x, agg2, deg4, wqc, bc, V):
    grid = (_N + _RB - 1) // _RB
    return pl.pallas_call(
        _dense_body,
        grid=(grid,),
        in_specs=[
            pl.BlockSpec((_RB, _D), lambda i: (i, 0)),
            pl.BlockSpec((_NC, _RB, _D), lambda i: (0, i, 0)),
            pl.BlockSpec((_NW, _QB, 1, _CHUNK), lambda i: (0, i, 0, 0)),
            pl.BlockSpec((_D, _M), lambda i: (0, 0)),
            pl.BlockSpec((1, _M), lambda i: (0, 0)),
            pl.BlockSpec((_M, _D, _D), lambda i: (0, 0, 0)),
        ],
        out_specs=pl.BlockSpec((_RB, _D), lambda i: (i, 0)),
        out_shape=jax.ShapeDtypeStruct((_N, _D), jnp.float32),
    )(x, agg2, deg4, wqc, bc, V)


def kernel(x, edge_index, Wq, bq, Wcode, V):
    src = edge_index[0]
    dst = edge_index[1]
    pad = _EPAD - _E
    src_p = jnp.concatenate(
        [src, jnp.zeros((pad,), jnp.int32)]).reshape(_NW, _CPW, _CHUNK)
    dump = _N + jnp.arange(pad, dtype=jnp.int32) % (_NPAD - _N)
    dst_p = jnp.concatenate([dst, dump]).reshape(_NW, _CPW, _CHUNK)
    zeros = jnp.zeros((_RPT, _D), jnp.float32)
    zflat = jnp.zeros((_NPAD,), jnp.float32)

    agg2, deg2 = _sc_agg()(src_p, dst_p, x, zeros, zflat)

    wqc = (Wq @ Wcode.T) / _TEMP
    bc = (bq[None, :] @ Wcode.T) / _TEMP

    deg4 = deg2.reshape(_NW, _DB, 1, _CHUNK)
    return _dense_call(x, agg2, deg4, wqc, bc, V)
